# Initial kernel scaffold; baseline (speedup 1.0000x reference)
import jax
import jax.numpy as jnp
from jax.experimental import pallas as pl


def kernel(x, edge_index, sampled_nodes, W1, W2):
    raise NotImplementedError("write your pallas kernel here")


# speedup vs baseline: 17.5402x; 17.5402x over previous
import functools

import jax
import jax.numpy as jnp
from jax import lax
from jax.experimental import pallas as pl
from jax.experimental.pallas import tpu as pltpu
from jax.experimental.pallas import tpu_sc as plsc

N = 10000
NP = 10240
E = 320000
BATCH = 128
NW = 32
BPW = 79
EPAD = NW * BPW * BATCH
BPT = 158
STRIPE = NP // 16
S = 1000
SP = 1024
SPT = SP // 16

_MESH = plsc.VectorSubcoreMesh(
    core_axis_name="c", subcore_axis_name="s", num_cores=2, num_subcores=16)
_SC_PARAMS = pltpu.CompilerParams(use_tc_tiling_on_sc=False)


def _deg_body(ed, zeros1, out, idx_v, ones_v, acc, sem):
    cid = lax.axis_index("c")
    sid = lax.axis_index("s")
    for i in range(8):
        ones_v[pl.ds(i * 16, 16)] = jnp.full((16,), 1.0, jnp.float32)
    pltpu.sync_copy(zeros1.at[pl.ds(sid * STRIPE, STRIPE)],
                    acc.at[pl.ds(sid * STRIPE, STRIPE)])
    plsc.subcore_barrier()
    pltpu.sync_copy(ed.at[cid, sid], idx_v)

    def body(j, carry):
        pltpu.sync_copy(ones_v, acc.at[idx_v.at[j]], add=True)
        return carry

    lax.fori_loop(0, BPT, body, 0)
    plsc.subcore_barrier()
    pltpu.sync_copy(acc.at[pl.ds(sid * STRIPE, STRIPE)],
                    out.at[cid, pl.ds(sid * STRIPE, STRIPE)])


def _prop_body(feat, srcw, dstw, zerosf, out, idxs, idxd, rows, acc, sem, *, F):
    cid = lax.axis_index("c")
    sid = lax.axis_index("s")
    wid = cid * 16 + sid
    pltpu.sync_copy(zerosf.at[pl.ds(sid * STRIPE, STRIPE)],
                    acc.at[pl.ds(sid * STRIPE, STRIPE)])
    plsc.subcore_barrier()
    pltpu.sync_copy(srcw.at[wid], idxs)
    pltpu.sync_copy(dstw.at[wid], idxd)

    def body(j, carry):
        pltpu.async_copy(feat.at[idxs.at[j]], rows, sem).wait()
        pltpu.sync_copy(rows, acc.at[idxd.at[j]], add=True)
        return carry

    lax.fori_loop(0, BPW, body, 0)
    plsc.subcore_barrier()
    pltpu.sync_copy(acc.at[pl.ds(sid * STRIPE, STRIPE)],
                    out.at[cid, pl.ds(sid * STRIPE, STRIPE)])


def _prop_gather_body(feat, srcw, dstw, zerosf, bcol, sampw,
                      gpart, bg, idxs, idxd, rows, acc, sidx, srows, brows, sem):
    cid = lax.axis_index("c")
    sid = lax.axis_index("s")
    wid = cid * 16 + sid
    pltpu.sync_copy(zerosf.at[pl.ds(sid * STRIPE, STRIPE)],
                    acc.at[pl.ds(sid * STRIPE, STRIPE)])
    plsc.subcore_barrier()
    pltpu.sync_copy(srcw.at[wid], idxs)
    pltpu.sync_copy(dstw.at[wid], idxd)

    def body(j, carry):
        pltpu.async_copy(feat.at[idxs.at[j]], rows, sem).wait()
        pltpu.sync_copy(rows, acc.at[idxd.at[j]], add=True)
        return carry

    lax.fori_loop(0, BPW, body, 0)
    plsc.subcore_barrier()
    pltpu.sync_copy(sampw.at[sid], sidx)
    pltpu.async_copy(acc.at[sidx], srows, sem).wait()
    pltpu.sync_copy(srows, gpart.at[cid, pl.ds(sid * SPT, SPT)])

    @pl.when(cid == 0)
    def _():
        pltpu.async_copy(bcol.at[sidx], brows, sem).wait()
        pltpu.sync_copy(brows, bg.at[pl.ds(sid * SPT, SPT)])


def _mm1_body(x_ref, w_ref, dego_ref, degi_ref, hp1_ref, a_ref, b_ref):
    a = lax.rsqrt(jnp.maximum(dego_ref[...], 1.0))
    b = lax.rsqrt(jnp.maximum(degi_ref[...], 1.0))
    mm = jnp.dot(x_ref[...], w_ref[...], preferred_element_type=jnp.float32)
    hp1_ref[...] = mm * a
    a_ref[...] = a
    b_ref[...] = b


def _mm2_body(pp_ref, a_ref, b_ref, w_ref, hp2_ref, bcol_ref):
    pp = pp_ref[...]
    b = b_ref[...]
    h = jnp.maximum((pp[0] + pp[1]) * b, 0.0)
    mm = jnp.dot(h, w_ref[...], preferred_element_type=jnp.float32)
    hp2_ref[...] = mm * a_ref[...]
    bcol_ref[...] = jnp.broadcast_to(b, b_ref.shape[:1] + (16,))


def _dec_body(gpb_ref, bgb_ref, gpa_ref, bga_ref, g_ref, c_ref):
    gpb = gpb_ref[...]
    zsb = (gpb[0] + gpb[1]) * bgb_ref[...]
    gpa = gpa_ref[...]
    zsa = ((gpa[0] + gpa[1]) * bga_ref[...])[:S]
    gram = lax.dot_general(zsb, zsa, (((1,), (1,)), ((), ())),
                           preferred_element_type=jnp.float32)
    sqb = jnp.sum(zsb * zsb, axis=1)
    sqa = jnp.sum(zsa * zsa, axis=1)
    d2 = jnp.maximum(sqb[:, None] + sqa[None, :] - 2.0 * gram, 0.0)
    g_ref[...] = gram
    c_ref[...] = jnp.sqrt(d2 + 1e-12)


def kernel(x, edge_index, sampled_nodes, W1, W2):
    f32 = jnp.float32
    src = edge_index[0]
    dst = edge_index[1]
    pad = EPAD - E
    srcw = jnp.pad(src, (0, pad)).reshape(NW, BPW, BATCH)
    dstw = jnp.pad(dst, (0, pad), constant_values=N).reshape(NW, BPW, BATCH)
    ed = jnp.stack([jnp.pad(dst, (0, pad), constant_values=N),
                    jnp.pad(src, (0, pad), constant_values=N)]).reshape(2, 16, BPT, BATCH)
    sampw = jnp.pad(sampled_nodes, (0, SP - S)).reshape(16, SPT)
    xp = jnp.pad(x, ((0, NP - N), (0, 0)))
    zeros1 = jnp.zeros((NP,), f32)
    zeros32 = jnp.zeros((NP, 32), f32)
    zeros16 = jnp.zeros((NP, 16), f32)

    degs = pl.kernel(
        _deg_body,
        out_type=jax.ShapeDtypeStruct((2, NP), f32),
        mesh=_MESH,
        compiler_params=_SC_PARAMS,
        scratch_types=[
            pltpu.VMEM((BPT, BATCH), jnp.int32),
            pltpu.VMEM((BATCH,), f32),
            pltpu.VMEM_SHARED((NP,), f32),
            pltpu.SemaphoreType.DMA,
        ],
    )(ed, zeros1)

    degi = degs[0].reshape(NP, 1)
    dego = degs[1].reshape(NP, 1)

    RB = 512
    grid = NP // RB
    hp1, a, b = pl.pallas_call(
        _mm1_body,
        grid=(grid,),
        in_specs=[
            pl.BlockSpec((RB, 128), lambda i: (i, 0)),
            pl.BlockSpec((128, 32), lambda i: (0, 0)),
            pl.BlockSpec((RB, 1), lambda i: (i, 0)),
            pl.BlockSpec((RB, 1), lambda i: (i, 0)),
        ],
        out_specs=[
            pl.BlockSpec((RB, 32), lambda i: (i, 0)),
            pl.BlockSpec((RB, 1), lambda i: (i, 0)),
            pl.BlockSpec((RB, 1), lambda i: (i, 0)),
        ],
        out_shape=[
            jax.ShapeDtypeStruct((NP, 32), f32),
            jax.ShapeDtypeStruct((NP, 1), f32),
            jax.ShapeDtypeStruct((NP, 1), f32),
        ],
    )(xp, W1, dego, degi)

    p32 = pl.kernel(
        functools.partial(_prop_body, F=32),
        out_type=jax.ShapeDtypeStruct((2, NP, 32), f32),
        mesh=_MESH,
        compiler_params=_SC_PARAMS,
        scratch_types=[
            pltpu.VMEM((BPW, BATCH), jnp.int32),
            pltpu.VMEM((BPW, BATCH), jnp.int32),
            pltpu.VMEM((BATCH, 32), f32),
            pltpu.VMEM_SHARED((NP, 32), f32),
            pltpu.SemaphoreType.DMA,
        ],
    )(hp1, srcw, dstw, zeros32)

    hp2, bcol = pl.pallas_call(
        _mm2_body,
        grid=(grid,),
        in_specs=[
            pl.BlockSpec((2, RB, 32), lambda i: (0, i, 0)),
            pl.BlockSpec((RB, 1), lambda i: (i, 0)),
            pl.BlockSpec((RB, 1), lambda i: (i, 0)),
            pl.BlockSpec((32, 16), lambda i: (0, 0)),
        ],
        out_specs=[
            pl.BlockSpec((RB, 16), lambda i: (i, 0)),
            pl.BlockSpec((RB, 16), lambda i: (i, 0)),
        ],
        out_shape=[
            jax.ShapeDtypeStruct((NP, 16), f32),
            jax.ShapeDtypeStruct((NP, 16), f32),
        ],
    )(p32, a, b, W2)

    gpart, bg = pl.kernel(
        _prop_gather_body,
        out_type=(jax.ShapeDtypeStruct((2, SP, 16), f32),
                  jax.ShapeDtypeStruct((SP, 16), f32)),
        mesh=_MESH,
        compiler_params=_SC_PARAMS,
        scratch_types=[
            pltpu.VMEM((BPW, BATCH), jnp.int32),
            pltpu.VMEM((BPW, BATCH), jnp.int32),
            pltpu.VMEM((BATCH, 16), f32),
            pltpu.VMEM_SHARED((NP, 16), f32),
            pltpu.VMEM((SPT,), jnp.int32),
            pltpu.VMEM((SPT, 16), f32),
            pltpu.VMEM((SPT, 16), f32),
            pltpu.SemaphoreType.DMA,
        ],
    )(hp2, srcw, dstw, zeros16, bcol, sampw)

    G, C = pl.pallas_call(
        _dec_body,
        grid=(S // 8,),
        in_specs=[
            pl.BlockSpec((2, 8, 16), lambda i: (0, i, 0)),
            pl.BlockSpec((8, 16), lambda i: (i, 0)),
            pl.BlockSpec((2, SP, 16), lambda i: (0, 0, 0)),
            pl.BlockSpec((SP, 16), lambda i: (0, 0)),
        ],
        out_specs=[
            pl.BlockSpec((8, S), lambda i: (i, 0)),
            pl.BlockSpec((8, S), lambda i: (i, 0)),
        ],
        out_shape=[
            jax.ShapeDtypeStruct((S, S), f32),
            jax.ShapeDtypeStruct((S, S), f32),
        ],
    )(gpart, bg, gpart, bg)

    return jnp.stack([G.reshape(-1), C.reshape(-1)])


# speedup vs baseline: 23.9395x; 1.3648x over previous
import jax
import jax.numpy as jnp
from jax import lax
from jax.experimental import pallas as pl
from jax.experimental.pallas import tpu as pltpu
from jax.experimental.pallas import tpu_sc as plsc

N = 10000
NP = 10240
E = 320000
BATCH = 128
NW = 32
BPW = 80
EPAD = NW * BPW * BATCH
BPT = 160
STRIPE = NP // 16
S = 1000
SP = 1024
SPT = SP // 16

_MESH = plsc.VectorSubcoreMesh(
    core_axis_name="c", subcore_axis_name="s", num_cores=2, num_subcores=16)
_SC_PARAMS = pltpu.CompilerParams(use_tc_tiling_on_sc=False)


def _deg_body(ed, zeros1, out, idx_v, ones_v, acc, sem):
    cid = lax.axis_index("c")
    sid = lax.axis_index("s")
    for i in range(8):
        ones_v[pl.ds(i * 16, 16)] = jnp.full((16,), 1.0, jnp.float32)
    pltpu.sync_copy(zeros1, acc.at[pl.ds(sid * STRIPE, STRIPE)])
    plsc.subcore_barrier()
    pltpu.sync_copy(ed.at[cid, sid], idx_v)

    def body(j, carry):
        pltpu.sync_copy(ones_v, acc.at[idx_v.at[j]], add=True)
        return carry

    lax.fori_loop(0, BPT, body, 0)
    plsc.subcore_barrier()
    pltpu.sync_copy(acc.at[pl.ds(sid * STRIPE, STRIPE)],
                    out.at[cid, pl.ds(sid * STRIPE, STRIPE)])


def _edge_sweep(feat, idxs, idxd, rows0, rows1, acc, sem0, sem1):
    pltpu.async_copy(feat.at[idxs.at[0]], rows0, sem0)
    pltpu.async_copy(feat.at[idxs.at[1]], rows1, sem1)

    def body(jj, carry):
        j0 = 2 * jj
        pltpu.make_async_copy(feat.at[idxs.at[j0]], rows0, sem0).wait()
        pltpu.sync_copy(rows0, acc.at[idxd.at[j0]], add=True)

        @pl.when(jj < BPW // 2 - 1)
        def _():
            pltpu.async_copy(feat.at[idxs.at[j0 + 2]], rows0, sem0)

        pltpu.make_async_copy(feat.at[idxs.at[j0 + 1]], rows1, sem1).wait()
        pltpu.sync_copy(rows1, acc.at[idxd.at[j0 + 1]], add=True)

        @pl.when(jj < BPW // 2 - 1)
        def _():
            pltpu.async_copy(feat.at[idxs.at[j0 + 3]], rows1, sem1)

        return carry

    lax.fori_loop(0, BPW // 2, body, 0)


def _prop_body(feat, srcw, dstw, zerosf, out,
               idxs, idxd, rows0, rows1, acc, sem0, sem1):
    cid = lax.axis_index("c")
    sid = lax.axis_index("s")
    wid = cid * 16 + sid
    pltpu.sync_copy(zerosf, acc.at[pl.ds(sid * STRIPE, STRIPE)])
    pltpu.sync_copy(srcw.at[wid], idxs)
    pltpu.sync_copy(dstw.at[wid], idxd)
    plsc.subcore_barrier()
    _edge_sweep(feat, idxs, idxd, rows0, rows1, acc, sem0, sem1)
    plsc.subcore_barrier()
    pltpu.sync_copy(acc.at[pl.ds(sid * STRIPE, STRIPE)],
                    out.at[cid, pl.ds(sid * STRIPE, STRIPE)])


def _prop_gather_body(feat, srcw, dstw, zerosf, bcol, sampw, gpart, bg,
                      idxs, idxd, rows0, rows1, acc, sidx, srows, brows,
                      sem0, sem1):
    cid = lax.axis_index("c")
    sid = lax.axis_index("s")
    wid = cid * 16 + sid
    pltpu.sync_copy(zerosf, acc.at[pl.ds(sid * STRIPE, STRIPE)])
    pltpu.sync_copy(srcw.at[wid], idxs)
    pltpu.sync_copy(dstw.at[wid], idxd)
    pltpu.sync_copy(sampw.at[sid], sidx)
    plsc.subcore_barrier()
    _edge_sweep(feat, idxs, idxd, rows0, rows1, acc, sem0, sem1)
    plsc.subcore_barrier()
    pltpu.async_copy(acc.at[sidx], srows, sem0).wait()
    pltpu.sync_copy(srows, gpart.at[cid, pl.ds(sid * SPT, SPT)])

    @pl.when(cid == 0)
    def _():
        pltpu.async_copy(bcol.at[sidx], brows, sem1).wait()
        pltpu.sync_copy(brows, bg.at[pl.ds(sid * SPT, SPT)])


def _mm1_body(x_ref, w_ref, dego_ref, degi_ref, hp1_ref, a_ref, b_ref):
    a = lax.rsqrt(jnp.maximum(dego_ref[...], 1.0))
    b = lax.rsqrt(jnp.maximum(degi_ref[...], 1.0))
    mm = jnp.dot(x_ref[...], w_ref[...], preferred_element_type=jnp.float32)
    hp1_ref[...] = mm * a
    a_ref[...] = a
    b_ref[...] = b


def _mm2_body(pp_ref, a_ref, b_ref, w_ref, hp2_ref, bcol_ref):
    pp = pp_ref[...]
    b = b_ref[...]
    h = jnp.maximum((pp[0] + pp[1]) * b, 0.0)
    mm = jnp.dot(h, w_ref[...], preferred_element_type=jnp.float32)
    hp2_ref[...] = mm * a_ref[...]
    bcol_ref[...] = jnp.broadcast_to(b, (b.shape[0], 16))


def _dec_body(gpb_ref, bgb_ref, gpa_ref, bga_ref, out_ref):
    gpb = gpb_ref[...]
    zsb = (gpb[0] + gpb[1]) * bgb_ref[...]
    gpa = gpa_ref[...]
    zsa = ((gpa[0] + gpa[1]) * bga_ref[...])[:S]
    gram = lax.dot_general(zsb, zsa, (((1,), (1,)), ((), ())),
                           preferred_element_type=jnp.float32)
    sqb = jnp.sum(zsb * zsb, axis=1)
    sqa = jnp.sum(zsa * zsa, axis=1)
    d2 = jnp.maximum(sqb[:, None] + sqa[None, :] - 2.0 * gram, 0.0)
    out_ref[0] = gram
    out_ref[1] = jnp.sqrt(d2 + 1e-12)


def kernel(x, edge_index, sampled_nodes, W1, W2):
    f32 = jnp.float32
    pad = EPAD - E
    srcp = jnp.pad(edge_index[0], (0, pad), constant_values=N)
    dstp = jnp.pad(edge_index[1], (0, pad), constant_values=N)
    srcw = srcp.reshape(NW, BPW, BATCH)
    dstw = dstp.reshape(NW, BPW, BATCH)
    ed = jnp.stack([dstp, srcp]).reshape(2, 16, BPT, BATCH)
    sampw = jnp.pad(sampled_nodes, (0, SP - S)).reshape(16, SPT)
    zeros1 = jnp.zeros((STRIPE,), f32)
    zeros32 = jnp.zeros((STRIPE, 32), f32)
    zeros16 = jnp.zeros((STRIPE, 16), f32)

    degs = pl.kernel(
        _deg_body,
        out_type=jax.ShapeDtypeStruct((2, NP), f32),
        mesh=_MESH,
        compiler_params=_SC_PARAMS,
        scratch_types=[
            pltpu.VMEM((BPT, BATCH), jnp.int32),
            pltpu.VMEM((BATCH,), f32),
            pltpu.VMEM_SHARED((NP,), f32),
            pltpu.SemaphoreType.DMA,
        ],
    )(ed, zeros1)

    degi = degs[0].reshape(NP, 1)
    dego = degs[1].reshape(NP, 1)

    RB = 400
    grid = N // RB
    hp1, a, b = pl.pallas_call(
        _mm1_body,
        grid=(grid,),
        in_specs=[
            pl.BlockSpec((RB, 128), lambda i: (i, 0)),
            pl.BlockSpec((128, 32), lambda i: (0, 0)),
            pl.BlockSpec((RB, 1), lambda i: (i, 0)),
            pl.BlockSpec((RB, 1), lambda i: (i, 0)),
        ],
        out_specs=[
            pl.BlockSpec((RB, 32), lambda i: (i, 0)),
            pl.BlockSpec((RB, 1), lambda i: (i, 0)),
            pl.BlockSpec((RB, 1), lambda i: (i, 0)),
        ],
        out_shape=[
            jax.ShapeDtypeStruct((NP, 32), f32),
            jax.ShapeDtypeStruct((NP, 1), f32),
            jax.ShapeDtypeStruct((NP, 1), f32),
        ],
    )(x, W1, dego, degi)

    p32 = pl.kernel(
        _prop_body,
        out_type=jax.ShapeDtypeStruct((2, NP, 32), f32),
        mesh=_MESH,
        compiler_params=_SC_PARAMS,
        scratch_types=[
            pltpu.VMEM((BPW, BATCH), jnp.int32),
            pltpu.VMEM((BPW, BATCH), jnp.int32),
            pltpu.VMEM((BATCH, 32), f32),
            pltpu.VMEM((BATCH, 32), f32),
            pltpu.VMEM_SHARED((NP, 32), f32),
            pltpu.SemaphoreType.DMA,
            pltpu.SemaphoreType.DMA,
        ],
    )(hp1, srcw, dstw, zeros32)

    hp2, bcol = pl.pallas_call(
        _mm2_body,
        grid=(grid,),
        in_specs=[
            pl.BlockSpec((2, RB, 32), lambda i: (0, i, 0)),
            pl.BlockSpec((RB, 1), lambda i: (i, 0)),
            pl.BlockSpec((RB, 1), lambda i: (i, 0)),
            pl.BlockSpec((32, 16), lambda i: (0, 0)),
        ],
        out_specs=[
            pl.BlockSpec((RB, 16), lambda i: (i, 0)),
            pl.BlockSpec((RB, 16), lambda i: (i, 0)),
        ],
        out_shape=[
            jax.ShapeDtypeStruct((NP, 16), f32),
            jax.ShapeDtypeStruct((NP, 16), f32),
        ],
    )(p32, a, b, W2)

    gpart, bg = pl.kernel(
        _prop_gather_body,
        out_type=(jax.ShapeDtypeStruct((2, SP, 16), f32),
                  jax.ShapeDtypeStruct((SP, 16), f32)),
        mesh=_MESH,
        compiler_params=_SC_PARAMS,
        scratch_types=[
            pltpu.VMEM((BPW, BATCH), jnp.int32),
            pltpu.VMEM((BPW, BATCH), jnp.int32),
            pltpu.VMEM((BATCH, 16), f32),
            pltpu.VMEM((BATCH, 16), f32),
            pltpu.VMEM_SHARED((NP, 16), f32),
            pltpu.VMEM((SPT,), jnp.int32),
            pltpu.VMEM((SPT, 16), f32),
            pltpu.VMEM((SPT, 16), f32),
            pltpu.SemaphoreType.DMA,
            pltpu.SemaphoreType.DMA,
        ],
    )(hp2, srcw, dstw, zeros16, bcol, sampw)

    RBD = 200
    out = pl.pallas_call(
        _dec_body,
        grid=(S // RBD,),
        in_specs=[
            pl.BlockSpec((2, RBD, 16), lambda i: (0, i, 0)),
            pl.BlockSpec((RBD, 16), lambda i: (i, 0)),
            pl.BlockSpec((2, SP, 16), lambda i: (0, 0, 0)),
            pl.BlockSpec((SP, 16), lambda i: (0, 0)),
        ],
        out_specs=pl.BlockSpec((2, RBD, S), lambda i: (0, i, 0)),
        out_shape=jax.ShapeDtypeStruct((2, S, S), f32),
    )(gpart, bg, gpart, bg)

    return out.reshape(2, S * S)


# speedup vs baseline: 26.3677x; 1.1014x over previous
import jax
import jax.numpy as jnp
from jax import lax
from jax.experimental import pallas as pl
from jax.experimental.pallas import tpu as pltpu
from jax.experimental.pallas import tpu_sc as plsc

N = 10000
NP = 10240
E = 320000
BATCH = 128
B0 = 96
B1 = 64
BASE1 = 16 * B0
TB = 2592
EPAD = TB * BATCH
BPT = TB // 16
STRIPE = NP // 16
S = 1000
SP = 1024
SPT = SP // 16

_MESH = plsc.VectorSubcoreMesh(
    core_axis_name="c", subcore_axis_name="s", num_cores=2, num_subcores=16)
_SC_PARAMS = pltpu.CompilerParams(use_tc_tiling_on_sc=False)


def _deg_body(srcb, dstb, zeros1, out, idx_v, ones_v, acc, sem):
    cid = lax.axis_index("c")
    sid = lax.axis_index("s")
    for i in range(8):
        ones_v[pl.ds(i * 16, 16)] = jnp.full((16,), 1.0, jnp.float32)
    pltpu.sync_copy(zeros1, acc.at[pl.ds(sid * STRIPE, STRIPE)])

    @pl.when(cid == 0)
    def _():
        pltpu.sync_copy(dstb.at[pl.ds(sid * BPT, BPT)], idx_v)

    @pl.when(cid == 1)
    def _():
        pltpu.sync_copy(srcb.at[pl.ds(sid * BPT, BPT)], idx_v)

    plsc.subcore_barrier()

    def body(j, carry):
        pltpu.sync_copy(ones_v, acc.at[idx_v.at[j]], add=True)
        return carry

    lax.fori_loop(0, BPT, body, 0)
    plsc.subcore_barrier()
    pltpu.sync_copy(acc.at[pl.ds(sid * STRIPE, STRIPE)],
                    out.at[cid, pl.ds(sid * STRIPE, STRIPE)])


def _edge_sweep(feat, idxs, idxd, rows0, rows1, acc, sem0, sem1, nb):
    pltpu.async_copy(feat.at[idxs.at[0]], rows0, sem0)
    pltpu.async_copy(feat.at[idxs.at[1]], rows1, sem1)

    def body(jj, carry):
        j0 = 2 * jj

        @pl.when(j0 < nb)
        def _():
            pltpu.make_async_copy(feat.at[idxs.at[j0]], rows0, sem0).wait()
            pltpu.sync_copy(rows0, acc.at[idxd.at[j0]], add=True)

            @pl.when(j0 + 2 < nb)
            def _():
                pltpu.async_copy(feat.at[idxs.at[j0 + 2]], rows0, sem0)

            pltpu.make_async_copy(feat.at[idxs.at[j0 + 1]], rows1, sem1).wait()
            pltpu.sync_copy(rows1, acc.at[idxd.at[j0 + 1]], add=True)

            @pl.when(j0 + 3 < nb)
            def _():
                pltpu.async_copy(feat.at[idxs.at[j0 + 3]], rows1, sem1)

        return carry

    lax.fori_loop(0, B0 // 2, body, 0)


def _sweep_prologue(cid, sid, srcb, dstb, idxs, idxd):
    base = jnp.where(cid == 0, sid * B0, BASE1 + sid * B1)
    nb = jnp.where(cid == 0, B0, B1)
    pltpu.sync_copy(srcb.at[pl.ds(base, B0)], idxs)
    pltpu.sync_copy(dstb.at[pl.ds(base, B0)], idxd)
    return nb


def _prop_body(feat, srcb, dstb, zerosf, out,
               idxs, idxd, rows0, rows1, acc, sem0, sem1):
    cid = lax.axis_index("c")
    sid = lax.axis_index("s")
    pltpu.sync_copy(zerosf, acc.at[pl.ds(sid * STRIPE, STRIPE)])
    nb = _sweep_prologue(cid, sid, srcb, dstb, idxs, idxd)
    plsc.subcore_barrier()
    _edge_sweep(feat, idxs, idxd, rows0, rows1, acc, sem0, sem1, nb)
    plsc.subcore_barrier()
    pltpu.sync_copy(acc.at[pl.ds(sid * STRIPE, STRIPE)],
                    out.at[cid, pl.ds(sid * STRIPE, STRIPE)])


def _prop_gather_body(feat, srcb, dstb, zerosf, bcol, sampw, gpart, bg,
                      idxs, idxd, rows0, rows1, acc, sidx, srows, brows,
                      sem0, sem1):
    cid = lax.axis_index("c")
    sid = lax.axis_index("s")
    pltpu.sync_copy(zerosf, acc.at[pl.ds(sid * STRIPE, STRIPE)])
    nb = _sweep_prologue(cid, sid, srcb, dstb, idxs, idxd)
    pltpu.sync_copy(sampw.at[sid], sidx)
    plsc.subcore_barrier()
    _edge_sweep(feat, idxs, idxd, rows0, rows1, acc, sem0, sem1, nb)
    plsc.subcore_barrier()
    pltpu.async_copy(acc.at[sidx], srows, sem0).wait()
    pltpu.sync_copy(srows, gpart.at[cid, pl.ds(sid * SPT, SPT)])

    @pl.when(cid == 0)
    def _():
        pltpu.async_copy(bcol.at[sidx], brows, sem1).wait()
        pltpu.sync_copy(brows, bg.at[pl.ds(sid * SPT, SPT)])


def _mm1_body(x_ref, w_ref, dego_ref, degi_ref, hp1_ref, a_ref, b_ref):
    a = lax.rsqrt(jnp.maximum(dego_ref[...], 1.0))
    b = lax.rsqrt(jnp.maximum(degi_ref[...], 1.0))
    mm = jnp.dot(x_ref[...], w_ref[...], preferred_element_type=jnp.float32)
    hp1_ref[...] = mm * a
    a_ref[...] = a
    b_ref[...] = b


def _mm2_body(pp_ref, a_ref, b_ref, w_ref, hp2_ref, bcol_ref):
    pp = pp_ref[...]
    b = b_ref[...]
    h = jnp.maximum((pp[0] + pp[1]) * b, 0.0)
    mm = jnp.dot(h, w_ref[...], preferred_element_type=jnp.float32)
    hp2_ref[...] = mm * a_ref[...]
    bcol_ref[...] = jnp.broadcast_to(b, (b.shape[0], 16))


def _dec_body(gpb_ref, bgb_ref, gpa_ref, bga_ref, out_ref):
    gpb = gpb_ref[...]
    zsb = (gpb[0] + gpb[1]) * bgb_ref[...]
    gpa = gpa_ref[...]
    zsa = ((gpa[0] + gpa[1]) * bga_ref[...])[:S]
    gram = lax.dot_general(zsb, zsa, (((1,), (1,)), ((), ())),
                           preferred_element_type=jnp.float32)
    sqb = jnp.sum(zsb * zsb, axis=1)
    sqa = jnp.sum(zsa * zsa, axis=1)
    d2 = jnp.maximum(sqb[:, None] + sqa[None, :] - 2.0 * gram, 0.0)
    out_ref[0] = gram
    out_ref[1] = jnp.sqrt(d2 + 1e-12)


def kernel(x, edge_index, sampled_nodes, W1, W2):
    f32 = jnp.float32
    pad = EPAD - E
    srcb = jnp.pad(edge_index[0], (0, pad), constant_values=N).reshape(TB, BATCH)
    dstb = jnp.pad(edge_index[1], (0, pad), constant_values=N).reshape(TB, BATCH)
    sampw = jnp.pad(sampled_nodes, (0, SP - S)).reshape(16, SPT)
    zeros1 = jnp.zeros((STRIPE,), f32)
    zeros32 = jnp.zeros((STRIPE, 32), f32)
    zeros16 = jnp.zeros((STRIPE, 16), f32)

    degs = pl.kernel(
        _deg_body,
        out_type=jax.ShapeDtypeStruct((2, NP), f32),
        mesh=_MESH,
        compiler_params=_SC_PARAMS,
        scratch_types=[
            pltpu.VMEM((BPT, BATCH), jnp.int32),
            pltpu.VMEM((BATCH,), f32),
            pltpu.VMEM_SHARED((NP,), f32),
            pltpu.SemaphoreType.DMA,
        ],
    )(srcb, dstb, zeros1)

    degi = degs[0].reshape(NP, 1)
    dego = degs[1].reshape(NP, 1)

    RB = 2000
    grid = N // RB
    hp1, a, b = pl.pallas_call(
        _mm1_body,
        grid=(grid,),
        in_specs=[
            pl.BlockSpec((RB, 128), lambda i: (i, 0)),
            pl.BlockSpec((128, 32), lambda i: (0, 0)),
            pl.BlockSpec((RB, 1), lambda i: (i, 0)),
            pl.BlockSpec((RB, 1), lambda i: (i, 0)),
        ],
        out_specs=[
            pl.BlockSpec((RB, 32), lambda i: (i, 0)),
            pl.BlockSpec((RB, 1), lambda i: (i, 0)),
            pl.BlockSpec((RB, 1), lambda i: (i, 0)),
        ],
        out_shape=[
            jax.ShapeDtypeStruct((NP, 32), f32),
            jax.ShapeDtypeStruct((NP, 1), f32),
            jax.ShapeDtypeStruct((NP, 1), f32),
        ],
    )(x, W1, dego, degi)

    p32 = pl.kernel(
        _prop_body,
        out_type=jax.ShapeDtypeStruct((2, NP, 32), f32),
        mesh=_MESH,
        compiler_params=_SC_PARAMS,
        scratch_types=[
            pltpu.VMEM((B0, BATCH), jnp.int32),
            pltpu.VMEM((B0, BATCH), jnp.int32),
            pltpu.VMEM((BATCH, 32), f32),
            pltpu.VMEM((BATCH, 32), f32),
            pltpu.VMEM_SHARED((NP, 32), f32),
            pltpu.SemaphoreType.DMA,
            pltpu.SemaphoreType.DMA,
        ],
    )(hp1, srcb, dstb, zeros32)

    hp2, bcol = pl.pallas_call(
        _mm2_body,
        grid=(grid,),
        in_specs=[
            pl.BlockSpec((2, RB, 32), lambda i: (0, i, 0)),
            pl.BlockSpec((RB, 1), lambda i: (i, 0)),
            pl.BlockSpec((RB, 1), lambda i: (i, 0)),
            pl.BlockSpec((32, 16), lambda i: (0, 0)),
        ],
        out_specs=[
            pl.BlockSpec((RB, 16), lambda i: (i, 0)),
            pl.BlockSpec((RB, 16), lambda i: (i, 0)),
        ],
        out_shape=[
            jax.ShapeDtypeStruct((NP, 16), f32),
            jax.ShapeDtypeStruct((NP, 16), f32),
        ],
    )(p32, a, b, W2)

    gpart, bg = pl.kernel(
        _prop_gather_body,
        out_type=(jax.ShapeDtypeStruct((2, SP, 16), f32),
                  jax.ShapeDtypeStruct((SP, 16), f32)),
        mesh=_MESH,
        compiler_params=_SC_PARAMS,
        scratch_types=[
            pltpu.VMEM((B0, BATCH), jnp.int32),
            pltpu.VMEM((B0, BATCH), jnp.int32),
            pltpu.VMEM((BATCH, 16), f32),
            pltpu.VMEM((BATCH, 16), f32),
            pltpu.VMEM_SHARED((NP, 16), f32),
            pltpu.VMEM((SPT,), jnp.int32),
            pltpu.VMEM((SPT, 16), f32),
            pltpu.VMEM((SPT, 16), f32),
            pltpu.SemaphoreType.DMA,
            pltpu.SemaphoreType.DMA,
        ],
    )(hp2, srcb, dstb, zeros16, bcol, sampw)

    RBD = 200
    out = pl.pallas_call(
        _dec_body,
        grid=(S // RBD,),
        in_specs=[
            pl.BlockSpec((2, RBD, 16), lambda i: (0, i, 0)),
            pl.BlockSpec((RBD, 16), lambda i: (i, 0)),
            pl.BlockSpec((2, SP, 16), lambda i: (0, 0, 0)),
            pl.BlockSpec((SP, 16), lambda i: (0, 0)),
        ],
        out_specs=pl.BlockSpec((2, RBD, S), lambda i: (0, i, 0)),
        out_shape=jax.ShapeDtypeStruct((2, S, S), f32),
    )(gpart, bg, gpart, bg)

    return out.reshape(2, S * S)


# speedup vs baseline: 36.7177x; 1.3925x over previous
import jax
import jax.numpy as jnp
from jax import lax
from jax.experimental import pallas as pl
from jax.experimental.pallas import tpu as pltpu
from jax.experimental.pallas import tpu_sc as plsc

N = 10000
NP = 10240
E = 320000
BATCH = 128
TB = E // BATCH
CAP = 80
DBPT = 157
STRIPE = NP // 16
S = 1000
SP = 1024
SPT = SP // 16

_MESH = plsc.VectorSubcoreMesh(
    core_axis_name="c", subcore_axis_name="s", num_cores=2, num_subcores=16)
_SC_PARAMS = pltpu.CompilerParams(use_tc_tiling_on_sc=False)


def _deg_body(ei3, zeros1, out, idx_v, ones_v, acc, sem):
    cid = lax.axis_index("c")
    sid = lax.axis_index("s")
    for i in range(8):
        ones_v[pl.ds(i * 16, 16)] = jnp.full((16,), 1.0, jnp.float32)
    pltpu.sync_copy(zeros1, acc.at[pl.ds(sid * STRIPE, STRIPE)])
    plane = jnp.where(cid == 0, 1, 0)
    base = 156 * sid + jnp.minimum(sid, 4)
    nb = jnp.where(sid < 4, 157, 156)

    @pl.when(sid < 4)
    def _():
        pltpu.sync_copy(ei3.at[plane, pl.ds(base, 157)], idx_v)

    @pl.when(sid >= 4)
    def _():
        pltpu.sync_copy(ei3.at[plane, pl.ds(base, 156)], idx_v.at[pl.ds(0, 156)])

    plsc.subcore_barrier()

    def body(j, carry):
        @pl.when(j < nb)
        def _():
            pltpu.sync_copy(ones_v, acc.at[idx_v.at[j]], add=True)
        return carry

    lax.fori_loop(0, DBPT, body, 0)
    plsc.subcore_barrier()
    pltpu.sync_copy(acc.at[pl.ds(sid * STRIPE, STRIPE)],
                    out.at[cid, pl.ds(sid * STRIPE, STRIPE)])


def _sweep_prologue(wid, ei3, idxs, idxd):
    base = 80 * jnp.minimum(wid, 2) + 78 * jnp.maximum(wid - 2, 0)
    nb = jnp.where(wid < 2, 80, 78)

    @pl.when(wid < 2)
    def _():
        pltpu.sync_copy(ei3.at[0, pl.ds(base, 80)], idxs)
        pltpu.sync_copy(ei3.at[1, pl.ds(base, 80)], idxd)

    @pl.when(wid >= 2)
    def _():
        pltpu.sync_copy(ei3.at[0, pl.ds(base, 78)], idxs.at[pl.ds(0, 78)])
        pltpu.sync_copy(ei3.at[1, pl.ds(base, 78)], idxd.at[pl.ds(0, 78)])

    return nb


def _edge_sweep(feat, idxs, idxd, rows0, rows1, acc, sem0, sem1, nb):
    pltpu.async_copy(feat.at[idxs.at[0]], rows0, sem0)
    pltpu.async_copy(feat.at[idxs.at[1]], rows1, sem1)

    def body(jj, carry):
        j0 = 2 * jj

        @pl.when(j0 < nb)
        def _():
            pltpu.make_async_copy(feat.at[idxs.at[j0]], rows0, sem0).wait()
            pltpu.sync_copy(rows0, acc.at[idxd.at[j0]], add=True)

            @pl.when(j0 + 2 < nb)
            def _():
                pltpu.async_copy(feat.at[idxs.at[j0 + 2]], rows0, sem0)

            pltpu.make_async_copy(feat.at[idxs.at[j0 + 1]], rows1, sem1).wait()
            pltpu.sync_copy(rows1, acc.at[idxd.at[j0 + 1]], add=True)

            @pl.when(j0 + 3 < nb)
            def _():
                pltpu.async_copy(feat.at[idxs.at[j0 + 3]], rows1, sem1)

        return carry

    lax.fori_loop(0, CAP // 2, body, 0)


def _prop_body(feat, ei3, zerosf, out,
               idxs, idxd, rows0, rows1, acc, sem0, sem1):
    cid = lax.axis_index("c")
    sid = lax.axis_index("s")
    wid = cid * 16 + sid
    pltpu.sync_copy(zerosf, acc.at[pl.ds(sid * STRIPE, STRIPE)])
    nb = _sweep_prologue(wid, ei3, idxs, idxd)
    plsc.subcore_barrier()
    _edge_sweep(feat, idxs, idxd, rows0, rows1, acc, sem0, sem1, nb)
    plsc.subcore_barrier()
    pltpu.sync_copy(acc.at[pl.ds(sid * STRIPE, STRIPE)],
                    out.at[cid, pl.ds(sid * STRIPE, STRIPE)])


def _prop_gather_body(feat, ei3, zerosf, bcol, sampw, gpart, bg,
                      idxs, idxd, rows0, rows1, acc, sidx, srows, brows,
                      sem0, sem1):
    cid = lax.axis_index("c")
    sid = lax.axis_index("s")
    wid = cid * 16 + sid
    pltpu.sync_copy(zerosf, acc.at[pl.ds(sid * STRIPE, STRIPE)])
    nb = _sweep_prologue(wid, ei3, idxs, idxd)
    pltpu.sync_copy(sampw.at[sid], sidx)
    plsc.subcore_barrier()
    _edge_sweep(feat, idxs, idxd, rows0, rows1, acc, sem0, sem1, nb)
    plsc.subcore_barrier()
    pltpu.async_copy(acc.at[sidx], srows, sem0).wait()
    pltpu.sync_copy(srows, gpart.at[cid, pl.ds(sid * SPT, SPT)])

    @pl.when(cid == 0)
    def _():
        pltpu.async_copy(bcol.at[sidx], brows, sem1).wait()
        pltpu.sync_copy(brows, bg.at[pl.ds(sid * SPT, SPT)])


def _mm1_body(x_ref, w_ref, deg_ref, hp1_ref, a_ref, b_ref):
    deg = deg_ref[...]
    a = lax.rsqrt(jnp.maximum(deg[1], 1.0)).reshape(-1, 1)
    b = lax.rsqrt(jnp.maximum(deg[0], 1.0)).reshape(-1, 1)
    mm = jnp.dot(x_ref[...], w_ref[...], preferred_element_type=jnp.float32)
    hp1_ref[...] = mm * a
    a_ref[...] = a
    b_ref[...] = b


def _mm2_body(pp_ref, a_ref, b_ref, w_ref, hp2_ref, bcol_ref):
    pp = pp_ref[...]
    b = b_ref[...]
    h = jnp.maximum((pp[0] + pp[1]) * b, 0.0)
    mm = jnp.dot(h, w_ref[...], preferred_element_type=jnp.float32)
    hp2_ref[...] = mm * a_ref[...]
    bcol_ref[...] = jnp.broadcast_to(b, (b.shape[0], 16))


def _dec_body(gpb_ref, bgb_ref, gpa_ref, bga_ref, out_ref):
    gpb = gpb_ref[...]
    zsb = (gpb[0] + gpb[1]) * bgb_ref[...]
    gpa = gpa_ref[...]
    zsa = ((gpa[0] + gpa[1]) * bga_ref[...])[:S]
    gram = lax.dot_general(zsb, zsa, (((1,), (1,)), ((), ())),
                           preferred_element_type=jnp.float32)
    sqb = jnp.sum(zsb * zsb, axis=1)
    sqa = jnp.sum(zsa * zsa, axis=1)
    d2 = jnp.maximum(sqb[:, None] + sqa[None, :] - 2.0 * gram, 0.0)
    out_ref[0] = gram
    out_ref[1] = jnp.sqrt(d2 + 1e-12)


def kernel(x, edge_index, sampled_nodes, W1, W2):
    f32 = jnp.float32
    ei3 = edge_index.reshape(2, TB, BATCH)
    sampw = jnp.pad(sampled_nodes, (0, SP - S)).reshape(16, SPT)
    zeros1 = jnp.zeros((STRIPE,), f32)
    zeros32 = jnp.zeros((STRIPE, 32), f32)
    zeros16 = jnp.zeros((STRIPE, 16), f32)

    degs = pl.kernel(
        _deg_body,
        out_type=jax.ShapeDtypeStruct((2, NP), f32),
        mesh=_MESH,
        compiler_params=_SC_PARAMS,
        scratch_types=[
            pltpu.VMEM((DBPT, BATCH), jnp.int32),
            pltpu.VMEM((BATCH,), f32),
            pltpu.VMEM_SHARED((NP,), f32),
            pltpu.SemaphoreType.DMA,
        ],
    )(ei3, zeros1)

    RB = 2048
    grid = NP // RB
    hp1, a, b = pl.pallas_call(
        _mm1_body,
        grid=(grid,),
        in_specs=[
            pl.BlockSpec((RB, 128), lambda i: (i, 0)),
            pl.BlockSpec((128, 32), lambda i: (0, 0)),
            pl.BlockSpec((2, RB), lambda i: (0, i)),
        ],
        out_specs=[
            pl.BlockSpec((RB, 32), lambda i: (i, 0)),
            pl.BlockSpec((RB, 1), lambda i: (i, 0)),
            pl.BlockSpec((RB, 1), lambda i: (i, 0)),
        ],
        out_shape=[
            jax.ShapeDtypeStruct((NP, 32), f32),
            jax.ShapeDtypeStruct((NP, 1), f32),
            jax.ShapeDtypeStruct((NP, 1), f32),
        ],
    )(x, W1, degs)

    p32 = pl.kernel(
        _prop_body,
        out_type=jax.ShapeDtypeStruct((2, NP, 32), f32),
        mesh=_MESH,
        compiler_params=_SC_PARAMS,
        scratch_types=[
            pltpu.VMEM((CAP, BATCH), jnp.int32),
            pltpu.VMEM((CAP, BATCH), jnp.int32),
            pltpu.VMEM((BATCH, 32), f32),
            pltpu.VMEM((BATCH, 32), f32),
            pltpu.VMEM_SHARED((NP, 32), f32),
            pltpu.SemaphoreType.DMA,
            pltpu.SemaphoreType.DMA,
        ],
    )(hp1, ei3, zeros32)

    hp2, bcol = pl.pallas_call(
        _mm2_body,
        grid=(grid,),
        in_specs=[
            pl.BlockSpec((2, RB, 32), lambda i: (0, i, 0)),
            pl.BlockSpec((RB, 1), lambda i: (i, 0)),
            pl.BlockSpec((RB, 1), lambda i: (i, 0)),
            pl.BlockSpec((32, 16), lambda i: (0, 0)),
        ],
        out_specs=[
            pl.BlockSpec((RB, 16), lambda i: (i, 0)),
            pl.BlockSpec((RB, 16), lambda i: (i, 0)),
        ],
        out_shape=[
            jax.ShapeDtypeStruct((NP, 16), f32),
            jax.ShapeDtypeStruct((NP, 16), f32),
        ],
    )(p32, a, b, W2)

    gpart, bg = pl.kernel(
        _prop_gather_body,
        out_type=(jax.ShapeDtypeStruct((2, SP, 16), f32),
                  jax.ShapeDtypeStruct((SP, 16), f32)),
        mesh=_MESH,
        compiler_params=_SC_PARAMS,
        scratch_types=[
            pltpu.VMEM((CAP, BATCH), jnp.int32),
            pltpu.VMEM((CAP, BATCH), jnp.int32),
            pltpu.VMEM((BATCH, 16), f32),
            pltpu.VMEM((BATCH, 16), f32),
            pltpu.VMEM_SHARED((NP, 16), f32),
            pltpu.VMEM((SPT,), jnp.int32),
            pltpu.VMEM((SPT, 16), f32),
            pltpu.VMEM((SPT, 16), f32),
            pltpu.SemaphoreType.DMA,
            pltpu.SemaphoreType.DMA,
        ],
    )(hp2, ei3, zeros16, bcol, sampw)

    RBD = 200
    out = pl.pallas_call(
        _dec_body,
        grid=(S // RBD,),
        in_specs=[
            pl.BlockSpec((2, RBD, 16), lambda i: (0, i, 0)),
            pl.BlockSpec((RBD, 16), lambda i: (i, 0)),
            pl.BlockSpec((2, SP, 16), lambda i: (0, 0, 0)),
            pl.BlockSpec((SP, 16), lambda i: (0, 0)),
        ],
        out_specs=pl.BlockSpec((2, RBD, S), lambda i: (0, i, 0)),
        out_shape=jax.ShapeDtypeStruct((2, S, S), f32),
    )(gpart, bg, gpart, bg)

    return out.reshape(2, S * S)


# speedup vs baseline: 43.2994x; 1.1793x over previous
import jax
import jax.numpy as jnp
from jax import lax
from jax.experimental import pallas as pl
from jax.experimental.pallas import tpu as pltpu
from jax.experimental.pallas import tpu_sc as plsc

N = 10000
NP = 10240
E = 320000
BATCH = 128
TB = E // BATCH
CAP = 80
DBPT = 157
STRIPE = NP // 16
S = 1000
SP = 1024
SPT = SP // 16

_MESH = plsc.VectorSubcoreMesh(
    core_axis_name="c", subcore_axis_name="s", num_cores=2, num_subcores=16)
_SC_PARAMS = pltpu.CompilerParams(use_tc_tiling_on_sc=False)


def _deg_body(ei3, zeros1, out, idx_v, ones_v, acc, sem):
    cid = lax.axis_index("c")
    sid = lax.axis_index("s")
    for i in range(8):
        ones_v[pl.ds(i * 16, 16)] = jnp.full((16,), 1.0, jnp.float32)
    pltpu.sync_copy(zeros1, acc.at[pl.ds(sid * STRIPE, STRIPE)])
    plane = jnp.where(cid == 0, 1, 0)
    base = 156 * sid + jnp.minimum(sid, 4)
    nb = jnp.where(sid < 4, 157, 156)

    @pl.when(sid < 4)
    def _():
        pltpu.sync_copy(ei3.at[plane, pl.ds(base, 157)], idx_v)

    @pl.when(sid >= 4)
    def _():
        pltpu.sync_copy(ei3.at[plane, pl.ds(base, 156)], idx_v.at[pl.ds(0, 156)])

    plsc.subcore_barrier()

    def body(j, carry):
        @pl.when(j < nb)
        def _():
            pltpu.sync_copy(ones_v, acc.at[idx_v.at[j]], add=True)
        return carry

    lax.fori_loop(0, DBPT, body, 0)
    plsc.subcore_barrier()
    pltpu.sync_copy(acc.at[pl.ds(sid * STRIPE, STRIPE)],
                    out.at[cid, pl.ds(sid * STRIPE, STRIPE)])


def _sweep_prologue(wid, ei3, idxs, idxd):
    base = 80 * jnp.minimum(wid, 17) + 76 * jnp.maximum(wid - 17, 0)
    nb = jnp.where(wid < 17, CAP, CAP - 4)

    @pl.when(wid < 17)
    def _():
        pltpu.sync_copy(ei3.at[0, pl.ds(base, CAP)], idxs)
        pltpu.sync_copy(ei3.at[1, pl.ds(base, CAP)], idxd)

    @pl.when(wid >= 17)
    def _():
        pltpu.sync_copy(ei3.at[0, pl.ds(base, CAP - 4)], idxs.at[pl.ds(0, CAP - 4)])
        pltpu.sync_copy(ei3.at[1, pl.ds(base, CAP - 4)], idxd.at[pl.ds(0, CAP - 4)])

    return nb


def _edge_sweep(feat, idxs, idxd, rows, acc, sems, nb):
    for b in range(4):
        pltpu.async_copy(feat.at[idxs.at[b]], rows[b], sems[b])

    def body(qq, carry):
        j0 = 4 * qq
        for b in range(4):
            j = j0 + b

            @pl.when(j < nb)
            def _(b=b, j=j):
                pltpu.make_async_copy(feat.at[idxs.at[j]], rows[b], sems[b]).wait()
                pltpu.sync_copy(rows[b], acc.at[idxd.at[j]], add=True)

                @pl.when(j + 4 < nb)
                def _():
                    pltpu.async_copy(feat.at[idxs.at[j + 4]], rows[b], sems[b])

        return carry

    lax.fori_loop(0, CAP // 4, body, 0)


def _prop_body(feat, ei3, zerosf, out,
               idxs, idxd, r0, r1, r2, r3, acc, s0, s1, s2, s3):
    cid = lax.axis_index("c")
    sid = lax.axis_index("s")
    wid = cid * 16 + sid
    pltpu.sync_copy(zerosf, acc.at[pl.ds(sid * STRIPE, STRIPE)])
    nb = _sweep_prologue(wid, ei3, idxs, idxd)
    plsc.subcore_barrier()
    _edge_sweep(feat, idxs, idxd, [r0, r1, r2, r3], acc, [s0, s1, s2, s3], nb)
    plsc.subcore_barrier()
    pltpu.sync_copy(acc.at[pl.ds(sid * STRIPE, STRIPE)],
                    out.at[cid, pl.ds(sid * STRIPE, STRIPE)])


def _prop_gather_body(feat, ei3, zerosf, bcol, sampw, gpart, bg,
                      idxs, idxd, r0, r1, r2, r3, acc, sidx, srows, brows,
                      s0, s1, s2, s3):
    cid = lax.axis_index("c")
    sid = lax.axis_index("s")
    wid = cid * 16 + sid
    pltpu.sync_copy(zerosf, acc.at[pl.ds(sid * STRIPE, STRIPE)])
    nb = _sweep_prologue(wid, ei3, idxs, idxd)
    pltpu.sync_copy(sampw.at[sid], sidx)
    plsc.subcore_barrier()
    _edge_sweep(feat, idxs, idxd, [r0, r1, r2, r3], acc, [s0, s1, s2, s3], nb)
    plsc.subcore_barrier()
    pltpu.async_copy(acc.at[sidx], srows, s0).wait()
    pltpu.sync_copy(srows, gpart.at[cid, pl.ds(sid * SPT, SPT)])

    @pl.when(cid == 0)
    def _():
        pltpu.async_copy(bcol.at[sidx], brows, s1).wait()
        pltpu.sync_copy(brows, bg.at[pl.ds(sid * SPT, SPT)])


def _mm1a_body(x_ref, w_ref, mm_ref):
    mm_ref[...] = jnp.dot(x_ref[...], w_ref[...],
                          preferred_element_type=jnp.float32)


def _mm1b_body(mm_ref, deg_ref, hp1_ref, a_ref, b_ref):
    deg = deg_ref[...]
    a = lax.rsqrt(jnp.maximum(deg[1], 1.0)).reshape(-1, 1)
    b = lax.rsqrt(jnp.maximum(deg[0], 1.0)).reshape(-1, 1)
    hp1_ref[...] = mm_ref[...] * a
    a_ref[...] = a
    b_ref[...] = b


def _mm2_body(pp_ref, a_ref, b_ref, w_ref, hp2_ref, bcol_ref):
    pp = pp_ref[...]
    b = b_ref[...]
    h = jnp.maximum((pp[0] + pp[1]) * b, 0.0)
    mm = jnp.dot(h, w_ref[...], preferred_element_type=jnp.float32)
    hp2_ref[...] = mm * a_ref[...]
    bcol_ref[...] = jnp.broadcast_to(b, (b.shape[0], 16))


def _dec_body(gpb_ref, bgb_ref, gpa_ref, bga_ref, out_ref):
    gpb = gpb_ref[...]
    zsb = (gpb[0] + gpb[1]) * bgb_ref[...]
    gpa = gpa_ref[...]
    zsa = ((gpa[0] + gpa[1]) * bga_ref[...])[:S]
    gram = lax.dot_general(zsb, zsa, (((1,), (1,)), ((), ())),
                           preferred_element_type=jnp.float32)
    sqb = jnp.sum(zsb * zsb, axis=1)
    sqa = jnp.sum(zsa * zsa, axis=1)
    d2 = jnp.maximum(sqb[:, None] + sqa[None, :] - 2.0 * gram, 0.0)
    out_ref[0] = gram
    out_ref[1] = jnp.sqrt(d2 + 1e-12)


def kernel(x, edge_index, sampled_nodes, W1, W2):
    f32 = jnp.float32
    ei3 = edge_index.reshape(2, TB, BATCH)
    sampw = jnp.pad(sampled_nodes, (0, SP - S)).reshape(16, SPT)
    zeros1 = jnp.zeros((STRIPE,), f32)
    zeros32 = jnp.zeros((STRIPE, 32), f32)
    zeros16 = jnp.zeros((STRIPE, 16), f32)

    degs = pl.kernel(
        _deg_body,
        out_type=jax.ShapeDtypeStruct((2, NP), f32),
        mesh=_MESH,
        compiler_params=_SC_PARAMS,
        scratch_types=[
            pltpu.VMEM((DBPT, BATCH), jnp.int32),
            pltpu.VMEM((BATCH,), f32),
            pltpu.VMEM_SHARED((NP,), f32),
            pltpu.SemaphoreType.DMA,
        ],
    )(ei3, zeros1)

    RB = 2048
    grid = NP // RB
    mmraw = pl.pallas_call(
        _mm1a_body,
        grid=(grid,),
        in_specs=[
            pl.BlockSpec((RB, 128), lambda i: (i, 0)),
            pl.BlockSpec((128, 32), lambda i: (0, 0)),
        ],
        out_specs=pl.BlockSpec((RB, 32), lambda i: (i, 0)),
        out_shape=jax.ShapeDtypeStruct((NP, 32), f32),
    )(x, W1)

    hp1, a, b = pl.pallas_call(
        _mm1b_body,
        grid=(grid,),
        in_specs=[
            pl.BlockSpec((RB, 32), lambda i: (i, 0)),
            pl.BlockSpec((2, RB), lambda i: (0, i)),
        ],
        out_specs=[
            pl.BlockSpec((RB, 32), lambda i: (i, 0)),
            pl.BlockSpec((RB, 1), lambda i: (i, 0)),
            pl.BlockSpec((RB, 1), lambda i: (i, 0)),
        ],
        out_shape=[
            jax.ShapeDtypeStruct((NP, 32), f32),
            jax.ShapeDtypeStruct((NP, 1), f32),
            jax.ShapeDtypeStruct((NP, 1), f32),
        ],
    )(mmraw, degs)

    p32 = pl.kernel(
        _prop_body,
        out_type=jax.ShapeDtypeStruct((2, NP, 32), f32),
        mesh=_MESH,
        compiler_params=_SC_PARAMS,
        scratch_types=[
            pltpu.VMEM((CAP, BATCH), jnp.int32),
            pltpu.VMEM((CAP, BATCH), jnp.int32),
            pltpu.VMEM((BATCH, 32), f32),
            pltpu.VMEM((BATCH, 32), f32),
            pltpu.VMEM((BATCH, 32), f32),
            pltpu.VMEM((BATCH, 32), f32),
            pltpu.VMEM_SHARED((NP, 32), f32),
            pltpu.SemaphoreType.DMA,
            pltpu.SemaphoreType.DMA,
            pltpu.SemaphoreType.DMA,
            pltpu.SemaphoreType.DMA,
        ],
    )(hp1, ei3, zeros32)

    hp2, bcol = pl.pallas_call(
        _mm2_body,
        grid=(grid,),
        in_specs=[
            pl.BlockSpec((2, RB, 32), lambda i: (0, i, 0)),
            pl.BlockSpec((RB, 1), lambda i: (i, 0)),
            pl.BlockSpec((RB, 1), lambda i: (i, 0)),
            pl.BlockSpec((32, 16), lambda i: (0, 0)),
        ],
        out_specs=[
            pl.BlockSpec((RB, 16), lambda i: (i, 0)),
            pl.BlockSpec((RB, 16), lambda i: (i, 0)),
        ],
        out_shape=[
            jax.ShapeDtypeStruct((NP, 16), f32),
            jax.ShapeDtypeStruct((NP, 16), f32),
        ],
    )(p32, a, b, W2)

    gpart, bg = pl.kernel(
        _prop_gather_body,
        out_type=(jax.ShapeDtypeStruct((2, SP, 16), f32),
                  jax.ShapeDtypeStruct((SP, 16), f32)),
        mesh=_MESH,
        compiler_params=_SC_PARAMS,
        scratch_types=[
            pltpu.VMEM((CAP, BATCH), jnp.int32),
            pltpu.VMEM((CAP, BATCH), jnp.int32),
            pltpu.VMEM((BATCH, 16), f32),
            pltpu.VMEM((BATCH, 16), f32),
            pltpu.VMEM((BATCH, 16), f32),
            pltpu.VMEM((BATCH, 16), f32),
            pltpu.VMEM_SHARED((NP, 16), f32),
            pltpu.VMEM((SPT,), jnp.int32),
            pltpu.VMEM((SPT, 16), f32),
            pltpu.VMEM((SPT, 16), f32),
            pltpu.SemaphoreType.DMA,
            pltpu.SemaphoreType.DMA,
            pltpu.SemaphoreType.DMA,
            pltpu.SemaphoreType.DMA,
        ],
    )(hp2, ei3, zeros16, bcol, sampw)

    RBD = 200
    out = pl.pallas_call(
        _dec_body,
        grid=(S // RBD,),
        in_specs=[
            pl.BlockSpec((2, RBD, 16), lambda i: (0, i, 0)),
            pl.BlockSpec((RBD, 16), lambda i: (i, 0)),
            pl.BlockSpec((2, SP, 16), lambda i: (0, 0, 0)),
            pl.BlockSpec((SP, 16), lambda i: (0, 0)),
        ],
        out_specs=pl.BlockSpec((2, RBD, S), lambda i: (0, i, 0)),
        out_shape=jax.ShapeDtypeStruct((2, S, S), f32),
    )(gpart, bg, gpart, bg)

    return out.reshape(2, S * S)


# speedup vs baseline: 48.7561x; 1.1260x over previous
import jax
import jax.numpy as jnp
from jax import lax
from jax.experimental import pallas as pl
from jax.experimental.pallas import tpu as pltpu
from jax.experimental.pallas import tpu_sc as plsc

N = 10000
NP = 10240
E = 320000
BATCH = 128
TB = E // BATCH
CAP = 80
DBPT = 157
STRIPE = NP // 16
S = 1000
SP = 1024
SPT = SP // 16

_MESH = plsc.VectorSubcoreMesh(
    core_axis_name="c", subcore_axis_name="s", num_cores=2, num_subcores=16)
_SC_PARAMS = pltpu.CompilerParams(use_tc_tiling_on_sc=False)


def _deg_body(ei3, zeros1, out, idx_v, ones_v, acc, sem):
    cid = lax.axis_index("c")
    sid = lax.axis_index("s")
    for i in range(8):
        ones_v[pl.ds(i * 16, 16)] = jnp.full((16,), 1.0, jnp.float32)
    pltpu.sync_copy(zeros1, acc.at[pl.ds(sid * STRIPE, STRIPE)])
    plane = jnp.where(cid == 0, 1, 0)
    base = 156 * sid + jnp.minimum(sid, 4)
    nb = jnp.where(sid < 4, 157, 156)

    @pl.when(sid < 4)
    def _():
        pltpu.sync_copy(ei3.at[plane, pl.ds(base, 157)], idx_v)

    @pl.when(sid >= 4)
    def _():
        pltpu.sync_copy(ei3.at[plane, pl.ds(base, 156)], idx_v.at[pl.ds(0, 156)])

    plsc.subcore_barrier()

    def body(j, carry):
        @pl.when(j < nb)
        def _():
            pltpu.async_copy(ones_v, acc.at[idx_v.at[j]], sem, add=True)
        return carry

    lax.fori_loop(0, DBPT, body, 0)

    def drain(j, carry):
        @pl.when(j < nb)
        def _():
            pltpu.make_async_copy(ones_v, acc.at[idx_v.at[j]], sem).wait()
        return carry

    lax.fori_loop(0, DBPT, drain, 0)
    plsc.subcore_barrier()
    pltpu.sync_copy(acc.at[pl.ds(sid * STRIPE, STRIPE)],
                    out.at[cid, pl.ds(sid * STRIPE, STRIPE)])


def _sweep_prologue(wid, ei3, idxs, idxd):
    base = 80 * jnp.minimum(wid, 17) + 76 * jnp.maximum(wid - 17, 0)
    nb = jnp.where(wid < 17, CAP, CAP - 4)

    @pl.when(wid < 17)
    def _():
        pltpu.sync_copy(ei3.at[0, pl.ds(base, CAP)], idxs)
        pltpu.sync_copy(ei3.at[1, pl.ds(base, CAP)], idxd)

    @pl.when(wid >= 17)
    def _():
        pltpu.sync_copy(ei3.at[0, pl.ds(base, CAP - 4)], idxs.at[pl.ds(0, CAP - 4)])
        pltpu.sync_copy(ei3.at[1, pl.ds(base, CAP - 4)], idxd.at[pl.ds(0, CAP - 4)])

    return nb


def _edge_sweep(feat, idxs, idxd, rows, acc, sems, nb):
    nd = len(rows)
    for b in range(nd):
        pltpu.async_copy(feat.at[idxs.at[b]], rows[b], sems[b])

    def body(qq, carry):
        j0 = nd * qq
        for b in range(nd):
            j = j0 + b

            @pl.when(j < nb)
            def _(b=b, j=j):
                pltpu.make_async_copy(feat.at[idxs.at[j]], rows[b], sems[b]).wait()
                pltpu.sync_copy(rows[b], acc.at[idxd.at[j]], add=True)

                @pl.when(j + nd < nb)
                def _():
                    pltpu.async_copy(feat.at[idxs.at[j + nd]], rows[b], sems[b])

        return carry

    lax.fori_loop(0, (CAP + nd - 1) // nd, body, 0)


def _prop_body(feat, ei3, zerosf, out, idxs, idxd,
               r0, r1, r2, r3, r4, r5, r6, r7, acc,
               s0, s1, s2, s3, s4, s5, s6, s7):
    cid = lax.axis_index("c")
    sid = lax.axis_index("s")
    wid = cid * 16 + sid
    pltpu.sync_copy(zerosf, acc.at[pl.ds(sid * STRIPE, STRIPE)])
    nb = _sweep_prologue(wid, ei3, idxs, idxd)
    plsc.subcore_barrier()
    _edge_sweep(feat, idxs, idxd, [r0, r1, r2, r3, r4, r5, r6, r7], acc,
                [s0, s1, s2, s3, s4, s5, s6, s7], nb)
    plsc.subcore_barrier()
    pltpu.sync_copy(acc.at[pl.ds(sid * STRIPE, STRIPE)],
                    out.at[cid, pl.ds(sid * STRIPE, STRIPE)])


def _prop_gather_body(feat, ei3, zerosf, bcol, sampw, gpart, bg, idxs, idxd,
                      r0, r1, r2, r3, r4, r5, r6, r7, acc, sidx, srows, brows,
                      s0, s1, s2, s3, s4, s5, s6, s7):
    cid = lax.axis_index("c")
    sid = lax.axis_index("s")
    wid = cid * 16 + sid
    pltpu.sync_copy(zerosf, acc.at[pl.ds(sid * STRIPE, STRIPE)])
    nb = _sweep_prologue(wid, ei3, idxs, idxd)
    pltpu.sync_copy(sampw.at[sid], sidx)
    plsc.subcore_barrier()
    _edge_sweep(feat, idxs, idxd, [r0, r1, r2, r3, r4, r5, r6, r7], acc,
                [s0, s1, s2, s3, s4, s5, s6, s7], nb)
    plsc.subcore_barrier()
    pltpu.async_copy(acc.at[sidx], srows, s0).wait()
    pltpu.sync_copy(srows, gpart.at[cid, pl.ds(sid * SPT, SPT)])

    @pl.when(cid == 0)
    def _():
        pltpu.async_copy(bcol.at[sidx], brows, s1).wait()
        pltpu.sync_copy(brows, bg.at[pl.ds(sid * SPT, SPT)])


def _mm1a_body(x_ref, w_ref, mm_ref):
    mm_ref[...] = jnp.dot(x_ref[...], w_ref[...],
                          preferred_element_type=jnp.float32)


def _mm1b_body(mm_ref, deg_ref, hp1_ref, a_ref, b_ref):
    deg = deg_ref[...]
    a = lax.rsqrt(jnp.maximum(deg[1], 1.0)).reshape(-1, 1)
    b = lax.rsqrt(jnp.maximum(deg[0], 1.0)).reshape(-1, 1)
    hp1_ref[...] = mm_ref[...] * a
    a_ref[...] = a
    b_ref[...] = b


def _mm2_body(pp_ref, a_ref, b_ref, w_ref, hp2_ref, bcol_ref):
    pp = pp_ref[...]
    b = b_ref[...]
    h = jnp.maximum((pp[0] + pp[1]) * b, 0.0)
    mm = jnp.dot(h, w_ref[...], preferred_element_type=jnp.float32)
    hp2_ref[...] = mm * a_ref[...]
    bcol_ref[...] = jnp.broadcast_to(b, (b.shape[0], 16))


def _dec_body(gpb_ref, bgb_ref, gpa_ref, bga_ref, out_ref):
    gpb = gpb_ref[...]
    zsb = (gpb[0] + gpb[1]) * bgb_ref[...]
    gpa = gpa_ref[...]
    zsa = ((gpa[0] + gpa[1]) * bga_ref[...])[:S]
    gram = lax.dot_general(zsb, zsa, (((1,), (1,)), ((), ())),
                           preferred_element_type=jnp.float32)
    sqb = jnp.sum(zsb * zsb, axis=1)
    sqa = jnp.sum(zsa * zsa, axis=1)
    d2 = jnp.maximum(sqb[:, None] + sqa[None, :] - 2.0 * gram, 0.0)
    out_ref[0] = gram
    out_ref[1] = jnp.sqrt(d2 + 1e-12)


def kernel(x, edge_index, sampled_nodes, W1, W2):
    f32 = jnp.float32
    ei3 = edge_index.reshape(2, TB, BATCH)
    sampw = jnp.pad(sampled_nodes, (0, SP - S)).reshape(16, SPT)
    zeros1 = jnp.zeros((STRIPE,), f32)
    zeros32 = jnp.zeros((STRIPE, 32), f32)
    zeros16 = jnp.zeros((STRIPE, 16), f32)

    degs = pl.kernel(
        _deg_body,
        out_type=jax.ShapeDtypeStruct((2, NP), f32),
        mesh=_MESH,
        compiler_params=_SC_PARAMS,
        scratch_types=[
            pltpu.VMEM((DBPT, BATCH), jnp.int32),
            pltpu.VMEM((BATCH,), f32),
            pltpu.VMEM_SHARED((NP,), f32),
            pltpu.SemaphoreType.DMA,
        ],
    )(ei3, zeros1)

    RB = 2048
    grid = NP // RB
    mmraw = pl.pallas_call(
        _mm1a_body,
        grid=(grid,),
        in_specs=[
            pl.BlockSpec((RB, 128), lambda i: (i, 0)),
            pl.BlockSpec((128, 32), lambda i: (0, 0)),
        ],
        out_specs=pl.BlockSpec((RB, 32), lambda i: (i, 0)),
        out_shape=jax.ShapeDtypeStruct((NP, 32), f32),
    )(x, W1)

    hp1, a, b = pl.pallas_call(
        _mm1b_body,
        grid=(grid,),
        in_specs=[
            pl.BlockSpec((RB, 32), lambda i: (i, 0)),
            pl.BlockSpec((2, RB), lambda i: (0, i)),
        ],
        out_specs=[
            pl.BlockSpec((RB, 32), lambda i: (i, 0)),
            pl.BlockSpec((RB, 1), lambda i: (i, 0)),
            pl.BlockSpec((RB, 1), lambda i: (i, 0)),
        ],
        out_shape=[
            jax.ShapeDtypeStruct((NP, 32), f32),
            jax.ShapeDtypeStruct((NP, 1), f32),
            jax.ShapeDtypeStruct((NP, 1), f32),
        ],
    )(mmraw, degs)

    p32 = pl.kernel(
        _prop_body,
        out_type=jax.ShapeDtypeStruct((2, NP, 32), f32),
        mesh=_MESH,
        compiler_params=_SC_PARAMS,
        scratch_types=[
            pltpu.VMEM((CAP, BATCH), jnp.int32),
            pltpu.VMEM((CAP, BATCH), jnp.int32),
            pltpu.VMEM((BATCH, 32), f32),
            pltpu.VMEM((BATCH, 32), f32),
            pltpu.VMEM((BATCH, 32), f32),
            pltpu.VMEM((BATCH, 32), f32),
            pltpu.VMEM((BATCH, 32), f32),
            pltpu.VMEM((BATCH, 32), f32),
            pltpu.VMEM((BATCH, 32), f32),
            pltpu.VMEM((BATCH, 32), f32),
            pltpu.VMEM_SHARED((NP, 32), f32),
        ] + [pltpu.SemaphoreType.DMA] * 8,
    )(hp1, ei3, zeros32)

    hp2, bcol = pl.pallas_call(
        _mm2_body,
        grid=(grid,),
        in_specs=[
            pl.BlockSpec((2, RB, 32), lambda i: (0, i, 0)),
            pl.BlockSpec((RB, 1), lambda i: (i, 0)),
            pl.BlockSpec((RB, 1), lambda i: (i, 0)),
            pl.BlockSpec((32, 16), lambda i: (0, 0)),
        ],
        out_specs=[
            pl.BlockSpec((RB, 16), lambda i: (i, 0)),
            pl.BlockSpec((RB, 16), lambda i: (i, 0)),
        ],
        out_shape=[
            jax.ShapeDtypeStruct((NP, 16), f32),
            jax.ShapeDtypeStruct((NP, 16), f32),
        ],
    )(p32, a, b, W2)

    gpart, bg = pl.kernel(
        _prop_gather_body,
        out_type=(jax.ShapeDtypeStruct((2, SP, 16), f32),
                  jax.ShapeDtypeStruct((SP, 16), f32)),
        mesh=_MESH,
        compiler_params=_SC_PARAMS,
        scratch_types=[
            pltpu.VMEM((CAP, BATCH), jnp.int32),
            pltpu.VMEM((CAP, BATCH), jnp.int32),
            pltpu.VMEM((BATCH, 16), f32),
            pltpu.VMEM((BATCH, 16), f32),
            pltpu.VMEM((BATCH, 16), f32),
            pltpu.VMEM((BATCH, 16), f32),
            pltpu.VMEM((BATCH, 16), f32),
            pltpu.VMEM((BATCH, 16), f32),
            pltpu.VMEM((BATCH, 16), f32),
            pltpu.VMEM((BATCH, 16), f32),
            pltpu.VMEM_SHARED((NP, 16), f32),
            pltpu.VMEM((SPT,), jnp.int32),
            pltpu.VMEM((SPT, 16), f32),
            pltpu.VMEM((SPT, 16), f32),
        ] + [pltpu.SemaphoreType.DMA] * 8,
    )(hp2, ei3, zeros16, bcol, sampw)

    RBD = 200
    out = pl.pallas_call(
        _dec_body,
        grid=(S // RBD,),
        in_specs=[
            pl.BlockSpec((2, RBD, 16), lambda i: (0, i, 0)),
            pl.BlockSpec((RBD, 16), lambda i: (i, 0)),
            pl.BlockSpec((2, SP, 16), lambda i: (0, 0, 0)),
            pl.BlockSpec((SP, 16), lambda i: (0, 0)),
        ],
        out_specs=pl.BlockSpec((2, RBD, S), lambda i: (0, i, 0)),
        out_shape=jax.ShapeDtypeStruct((2, S, S), f32),
    )(gpart, bg, gpart, bg)

    return out.reshape(2, S * S)


# speedup vs baseline: 50.0640x; 1.0268x over previous
import jax
import jax.numpy as jnp
from jax import lax
from jax.experimental import pallas as pl
from jax.experimental.pallas import tpu as pltpu
from jax.experimental.pallas import tpu_sc as plsc

N = 10000
NP = 10240
E = 320000
BATCH = 128
TB = E // BATCH
CAP = 80
DBPT = 157
STRIPE = NP // 16
S = 1000
SP = 1024
SPT = SP // 16

_MESH = plsc.VectorSubcoreMesh(
    core_axis_name="c", subcore_axis_name="s", num_cores=2, num_subcores=16)
_SC_PARAMS = pltpu.CompilerParams(use_tc_tiling_on_sc=False)


def _deg_body(ei3, zeros1, out, idx_v, ones_v, acc, sem):
    cid = lax.axis_index("c")
    sid = lax.axis_index("s")
    for i in range(8):
        ones_v[pl.ds(i * 16, 16)] = jnp.full((16,), 1.0, jnp.float32)
    pltpu.sync_copy(zeros1, acc.at[pl.ds(sid * STRIPE, STRIPE)])
    plane = jnp.where(cid == 0, 1, 0)
    base = 156 * sid + jnp.minimum(sid, 4)
    nb = jnp.where(sid < 4, 157, 156)

    @pl.when(sid < 4)
    def _():
        pltpu.sync_copy(ei3.at[plane, pl.ds(base, 157)], idx_v)

    @pl.when(sid >= 4)
    def _():
        pltpu.sync_copy(ei3.at[plane, pl.ds(base, 156)], idx_v.at[pl.ds(0, 156)])

    plsc.subcore_barrier()

    def body(j, carry):
        @pl.when(j < nb)
        def _():
            pltpu.async_copy(ones_v, acc.at[idx_v.at[j]], sem, add=True)
        return carry

    lax.fori_loop(0, DBPT, body, 0)

    def drain(j, carry):
        @pl.when(j < nb)
        def _():
            pltpu.make_async_copy(ones_v, acc.at[idx_v.at[j]], sem).wait()
        return carry

    lax.fori_loop(0, DBPT, drain, 0)
    plsc.subcore_barrier()
    pltpu.sync_copy(acc.at[pl.ds(sid * STRIPE, STRIPE)],
                    out.at[cid, pl.ds(sid * STRIPE, STRIPE)])


def _sweep_prologue(wid, ei3, idxs, idxd):
    base = 80 * jnp.minimum(wid, 17) + 76 * jnp.maximum(wid - 17, 0)
    nb = jnp.where(wid < 17, CAP, CAP - 4)

    @pl.when(wid < 17)
    def _():
        pltpu.sync_copy(ei3.at[0, pl.ds(base, CAP)], idxs)
        pltpu.sync_copy(ei3.at[1, pl.ds(base, CAP)], idxd)

    @pl.when(wid >= 17)
    def _():
        pltpu.sync_copy(ei3.at[0, pl.ds(base, CAP - 4)], idxs.at[pl.ds(0, CAP - 4)])
        pltpu.sync_copy(ei3.at[1, pl.ds(base, CAP - 4)], idxd.at[pl.ds(0, CAP - 4)])

    return nb


def _edge_sweep(feat, idxs, idxd, rows, acc, sems, nb):
    nd = len(rows)
    for b in range(nd):
        pltpu.async_copy(feat.at[idxs.at[b]], rows[b], sems[b])

    def body(qq, carry):
        j0 = nd * qq
        for b in range(nd):
            j = j0 + b

            @pl.when(j < nb)
            def _(b=b, j=j):
                pltpu.make_async_copy(feat.at[idxs.at[j]], rows[b], sems[b]).wait()
                pltpu.sync_copy(rows[b], acc.at[idxd.at[j]], add=True)

                @pl.when(j + nd < nb)
                def _():
                    pltpu.async_copy(feat.at[idxs.at[j + nd]], rows[b], sems[b])

        return carry

    lax.fori_loop(0, (CAP + nd - 1) // nd, body, 0)


def _prop_body(feat, ei3, zerosf, out, idxs, idxd,
               r0, r1, r2, r3, r4, r5, r6, r7, acc,
               s0, s1, s2, s3, s4, s5, s6, s7):
    cid = lax.axis_index("c")
    sid = lax.axis_index("s")
    wid = cid * 16 + sid
    pltpu.sync_copy(zerosf, acc.at[pl.ds(sid * STRIPE, STRIPE)])
    nb = _sweep_prologue(wid, ei3, idxs, idxd)
    plsc.subcore_barrier()
    _edge_sweep(feat, idxs, idxd, [r0, r1, r2, r3, r4, r5, r6, r7], acc,
                [s0, s1, s2, s3, s4, s5, s6, s7], nb)
    plsc.subcore_barrier()
    pltpu.sync_copy(acc.at[pl.ds(sid * STRIPE, STRIPE)],
                    out.at[cid, pl.ds(sid * STRIPE, STRIPE)])


def _prop_gather_body(feat, ei3, zerosf, bcol, sampw, gpart, bg, idxs, idxd,
                      r0, r1, r2, r3, r4, r5, r6, r7, acc, sidx, srows, brows,
                      s0, s1, s2, s3, s4, s5, s6, s7):
    cid = lax.axis_index("c")
    sid = lax.axis_index("s")
    wid = cid * 16 + sid
    pltpu.sync_copy(zerosf, acc.at[pl.ds(sid * STRIPE, STRIPE)])
    nb = _sweep_prologue(wid, ei3, idxs, idxd)
    pltpu.sync_copy(sampw.at[sid], sidx)
    plsc.subcore_barrier()
    _edge_sweep(feat, idxs, idxd, [r0, r1, r2, r3, r4, r5, r6, r7], acc,
                [s0, s1, s2, s3, s4, s5, s6, s7], nb)
    plsc.subcore_barrier()
    pltpu.async_copy(acc.at[sidx], srows, s0).wait()
    pltpu.sync_copy(srows, gpart.at[cid, pl.ds(sid * SPT, SPT)])

    @pl.when(cid == 0)
    def _():
        pltpu.async_copy(bcol.at[sidx], brows, s1).wait()
        pltpu.sync_copy(brows, bg.at[pl.ds(sid * SPT, SPT)])


def _mm1a_body(x_ref, w_ref, mm_ref):
    mm_ref[...] = jnp.dot(x_ref[...], w_ref[...],
                          preferred_element_type=jnp.float32)


def _mm1b_body(mm_ref, deg_ref, hp1_ref, a_ref, b_ref, bcol_ref):
    deg = deg_ref[...]
    a = lax.rsqrt(jnp.maximum(deg[1], 1.0)).reshape(-1, 1)
    b = lax.rsqrt(jnp.maximum(deg[0], 1.0)).reshape(-1, 1)
    hp1_ref[...] = mm_ref[...] * a
    a_ref[...] = a
    b_ref[...] = b
    bcol_ref[...] = jnp.broadcast_to(b, (b.shape[0], 16))


def _mm2_body(pp_ref, a_ref, b_ref, w_ref, hp2_ref):
    pp = pp_ref[...]
    h = jnp.maximum((pp[0] + pp[1]) * b_ref[...], 0.0)
    mm = jnp.dot(h, w_ref[...], preferred_element_type=jnp.float32)
    hp2_ref[...] = mm * a_ref[...]


def _dec_body(gpb_ref, bgb_ref, gpa_ref, bga_ref, out_ref):
    gpb = gpb_ref[...]
    zsb = (gpb[0] + gpb[1]) * bgb_ref[...]
    gpa = gpa_ref[...]
    zsa = ((gpa[0] + gpa[1]) * bga_ref[...])[:S]
    gram = lax.dot_general(zsb, zsa, (((1,), (1,)), ((), ())),
                           preferred_element_type=jnp.float32)
    sqb = jnp.sum(zsb * zsb, axis=1)
    sqa = jnp.sum(zsa * zsa, axis=1)
    d2 = jnp.maximum(sqb[:, None] + sqa[None, :] - 2.0 * gram, 0.0)
    out_ref[0] = gram
    out_ref[1] = jnp.sqrt(d2 + 1e-12)


def kernel(x, edge_index, sampled_nodes, W1, W2):
    f32 = jnp.float32
    ei3 = edge_index.reshape(2, TB, BATCH)
    sampw = jnp.pad(sampled_nodes, (0, SP - S)).reshape(16, SPT)
    zeros1 = jnp.zeros((STRIPE,), f32)
    zeros32 = jnp.zeros((STRIPE, 32), f32)
    zeros16 = jnp.zeros((STRIPE, 16), f32)

    degs = pl.kernel(
        _deg_body,
        out_type=jax.ShapeDtypeStruct((2, NP), f32),
        mesh=_MESH,
        compiler_params=_SC_PARAMS,
        scratch_types=[
            pltpu.VMEM((DBPT, BATCH), jnp.int32),
            pltpu.VMEM((BATCH,), f32),
            pltpu.VMEM_SHARED((NP,), f32),
            pltpu.SemaphoreType.DMA,
        ],
    )(ei3, zeros1)

    RB = 2048
    grid = NP // RB
    mmraw = pl.pallas_call(
        _mm1a_body,
        grid=(grid,),
        in_specs=[
            pl.BlockSpec((RB, 128), lambda i: (i, 0)),
            pl.BlockSpec((128, 32), lambda i: (0, 0)),
        ],
        out_specs=pl.BlockSpec((RB, 32), lambda i: (i, 0)),
        out_shape=jax.ShapeDtypeStruct((NP, 32), f32),
    )(x, W1)

    RB2 = 5120
    grid2 = NP // RB2
    hp1, a, b, bcol = pl.pallas_call(
        _mm1b_body,
        grid=(grid2,),
        in_specs=[
            pl.BlockSpec((RB2, 32), lambda i: (i, 0)),
            pl.BlockSpec((2, RB2), lambda i: (0, i)),
        ],
        out_specs=[
            pl.BlockSpec((RB2, 32), lambda i: (i, 0)),
            pl.BlockSpec((RB2, 1), lambda i: (i, 0)),
            pl.BlockSpec((RB2, 1), lambda i: (i, 0)),
            pl.BlockSpec((RB2, 16), lambda i: (i, 0)),
        ],
        out_shape=[
            jax.ShapeDtypeStruct((NP, 32), f32),
            jax.ShapeDtypeStruct((NP, 1), f32),
            jax.ShapeDtypeStruct((NP, 1), f32),
            jax.ShapeDtypeStruct((NP, 16), f32),
        ],
    )(mmraw, degs)

    p32 = pl.kernel(
        _prop_body,
        out_type=jax.ShapeDtypeStruct((2, NP, 32), f32),
        mesh=_MESH,
        compiler_params=_SC_PARAMS,
        scratch_types=[
            pltpu.VMEM((CAP, BATCH), jnp.int32),
            pltpu.VMEM((CAP, BATCH), jnp.int32),
            pltpu.VMEM((BATCH, 32), f32),
            pltpu.VMEM((BATCH, 32), f32),
            pltpu.VMEM((BATCH, 32), f32),
            pltpu.VMEM((BATCH, 32), f32),
            pltpu.VMEM((BATCH, 32), f32),
            pltpu.VMEM((BATCH, 32), f32),
            pltpu.VMEM((BATCH, 32), f32),
            pltpu.VMEM((BATCH, 32), f32),
            pltpu.VMEM_SHARED((NP, 32), f32),
        ] + [pltpu.SemaphoreType.DMA] * 8,
    )(hp1, ei3, zeros32)

    hp2 = pl.pallas_call(
        _mm2_body,
        grid=(grid2,),
        in_specs=[
            pl.BlockSpec((2, RB2, 32), lambda i: (0, i, 0)),
            pl.BlockSpec((RB2, 1), lambda i: (i, 0)),
            pl.BlockSpec((RB2, 1), lambda i: (i, 0)),
            pl.BlockSpec((32, 16), lambda i: (0, 0)),
        ],
        out_specs=pl.BlockSpec((RB2, 16), lambda i: (i, 0)),
        out_shape=jax.ShapeDtypeStruct((NP, 16), f32),
    )(p32, a, b, W2)

    gpart, bg = pl.kernel(
        _prop_gather_body,
        out_type=(jax.ShapeDtypeStruct((2, SP, 16), f32),
                  jax.ShapeDtypeStruct((SP, 16), f32)),
        mesh=_MESH,
        compiler_params=_SC_PARAMS,
        scratch_types=[
            pltpu.VMEM((CAP, BATCH), jnp.int32),
            pltpu.VMEM((CAP, BATCH), jnp.int32),
            pltpu.VMEM((BATCH, 16), f32),
            pltpu.VMEM((BATCH, 16), f32),
            pltpu.VMEM((BATCH, 16), f32),
            pltpu.VMEM((BATCH, 16), f32),
            pltpu.VMEM((BATCH, 16), f32),
            pltpu.VMEM((BATCH, 16), f32),
            pltpu.VMEM((BATCH, 16), f32),
            pltpu.VMEM((BATCH, 16), f32),
            pltpu.VMEM_SHARED((NP, 16), f32),
            pltpu.VMEM((SPT,), jnp.int32),
            pltpu.VMEM((SPT, 16), f32),
            pltpu.VMEM((SPT, 16), f32),
        ] + [pltpu.SemaphoreType.DMA] * 8,
    )(hp2, ei3, zeros16, bcol, sampw)

    RBD = 200
    out = pl.pallas_call(
        _dec_body,
        grid=(S // RBD,),
        in_specs=[
            pl.BlockSpec((2, RBD, 16), lambda i: (0, i, 0)),
            pl.BlockSpec((RBD, 16), lambda i: (i, 0)),
            pl.BlockSpec((2, SP, 16), lambda i: (0, 0, 0)),
            pl.BlockSpec((SP, 16), lambda i: (0, 0)),
        ],
        out_specs=pl.BlockSpec((2, RBD, S), lambda i: (0, i, 0)),
        out_shape=jax.ShapeDtypeStruct((2, S, S), f32),
    )(gpart, bg, gpart, bg)

    return out.reshape(2, S * S)
